# Initial kernel scaffold; baseline (speedup 1.0000x reference)
import jax
import jax.numpy as jnp
from jax.experimental import pallas as pl


def kernel(z, codebook):
    raise NotImplementedError("write your pallas kernel here")


# speedup vs baseline: 1.3299x; 1.3299x over previous
import functools

import jax
import jax.numpy as jnp
from jax.experimental import pallas as pl
from jax.experimental.pallas import tpu as pltpu
from jax.experimental.pallas import tpu_sc as plsc

_BN = 256
_GATHER_W = 128


def _csq_body(cb_ref, csq_ref):
    cb = cb_ref[...]
    csq_ref[...] = jnp.sum(cb * cb, axis=1)


def _dist_body(z_ref, cb_ref, csq_ref, idx_ref, mind2_ref):
    z = z_ref[...]
    zsq = jnp.sum(z * z, axis=1, keepdims=True)
    dot = jax.lax.dot_general(
        z * -2.0, cb_ref[...], (((1,), (1,)), ((), ())),
        preferred_element_type=jnp.float32)
    d2 = (zsq + dot) + csq_ref[...][None, :]
    m = jnp.min(d2, axis=1, keepdims=True)
    k = d2.shape[1]
    iota = jax.lax.broadcasted_iota(jnp.int32, d2.shape, 1)
    idx_ref[...] = jnp.min(jnp.where(d2 == m, iota, k), axis=1)
    mind2_ref[...] = m[:, 0]


def _loss_body(mind2_ref, loss_ref, *, denom):
    loss_ref[...] = (jnp.sum(mind2_ref[...]) / denom).reshape(1, 1)


def _tc_dist_argmin(z, codebook):
    n, d = z.shape
    k = codebook.shape[0]
    csq = pl.pallas_call(
        _csq_body,
        out_shape=jax.ShapeDtypeStruct((k,), jnp.float32),
    )(codebook)
    nb = n // _BN
    idx, mind2 = pl.pallas_call(
        _dist_body,
        grid=(nb,),
        in_specs=[
            pl.BlockSpec((_BN, d), lambda i: (i, 0)),
            pl.BlockSpec((k, d), lambda i: (0, 0)),
            pl.BlockSpec((k,), lambda i: (0,)),
        ],
        out_specs=[
            pl.BlockSpec((_BN,), lambda i: (i,)),
            pl.BlockSpec((_BN,), lambda i: (i,)),
        ],
        out_shape=[
            jax.ShapeDtypeStruct((n,), jnp.int32),
            jax.ShapeDtypeStruct((n,), jnp.float32),
        ],
        compiler_params=pltpu.CompilerParams(
            dimension_semantics=("parallel",)),
    )(z, codebook, csq)
    losssum = pl.pallas_call(
        functools.partial(_loss_body, denom=float(n * d)),
        out_shape=jax.ShapeDtypeStruct((1, 1), jnp.float32),
    )(mind2)
    return idx, losssum


def _sc_gather(codebook, indices):
    n = indices.shape[0]
    d = codebook.shape[1]
    idx2 = indices.reshape(1, n)
    mesh = plsc.VectorSubcoreMesh(
        core_axis_name="core", subcore_axis_name="subcore")

    @functools.partial(
        pl.kernel,
        out_type=jax.ShapeDtypeStruct((n, d), codebook.dtype),
        mesh=mesh)
    def gather_kernel(cb_hbm, i_hbm, o_hbm):
        def body(i_vmem, o_vmem):
            pltpu.sync_copy(cb_hbm.at[i_vmem.at[0]], o_vmem)

        pltpu.emit_pipeline(
            body,
            grid=(n // _GATHER_W,),
            in_specs=[pl.BlockSpec((1, _GATHER_W), index_map=lambda i: (0, i))],
            out_specs=[pl.BlockSpec((_GATHER_W, d), index_map=lambda i: (i, 0))],
            core_axis_name=("core", "subcore"),
            dimension_semantics=(pltpu.PARALLEL,),
        )(i_hbm, o_hbm)

    return gather_kernel(codebook, idx2)


def kernel(z, codebook):
    idx, losssum = _tc_dist_argmin(z, codebook)
    d = codebook.shape[1]
    if d < 128:
        cb_wide = jnp.pad(codebook, ((0, 0), (0, 128 - d)))
        quantized = _sc_gather(cb_wide, idx)[:, :d]
    else:
        quantized = _sc_gather(codebook, idx)
    return (quantized, idx, losssum[0, 0])


# speedup vs baseline: 1.7058x; 1.2826x over previous
import functools

import jax
import jax.numpy as jnp
from jax.experimental import pallas as pl
from jax.experimental.pallas import tpu as pltpu
from jax.experimental.pallas import tpu_sc as plsc

_BN = 256
_GATHER_W = 128


def _csq_body(cb_ref, csq_ref):
    cb = cb_ref[...]
    csq_ref[...] = jnp.sum(cb * cb, axis=1)


def _dist_body(z_ref, cb_ref, csq_ref, idx_ref, mind2_ref):
    z = z_ref[...]
    zsq = jnp.sum(z * z, axis=1, keepdims=True)
    dot = jax.lax.dot_general(
        z * -2.0, cb_ref[...], (((1,), (1,)), ((), ())),
        preferred_element_type=jnp.float32)
    k = dot.shape[1]
    csq = csq_ref[...]
    nl = 128
    bn = z.shape[0]
    lane_iota = jax.lax.broadcasted_iota(jnp.int32, (bn, nl), 1)
    run_min = jnp.full((bn, nl), jnp.inf, jnp.float32)
    run_idx = jnp.zeros((bn, nl), jnp.int32)
    for j in range(k // nl):
        d2j = (zsq + dot[:, j * nl:(j + 1) * nl]) + csq[j * nl:(j + 1) * nl][None, :]
        mask = d2j < run_min
        run_min = jnp.minimum(run_min, d2j)
        run_idx = jnp.where(mask, lane_iota + (j * nl), run_idx)
    m = jnp.min(run_min, axis=1, keepdims=True)
    tie = run_min == m
    idx_ref[...] = jnp.min(jnp.where(tie, run_idx, k), axis=1)
    mind2_ref[...] = m[:, 0]


def _loss_body(mind2_ref, loss_ref, *, denom):
    loss_ref[...] = (jnp.sum(mind2_ref[...]) / denom).reshape(1, 1)


def _tc_dist_argmin(z, codebook):
    n, d = z.shape
    k = codebook.shape[0]
    csq = pl.pallas_call(
        _csq_body,
        out_shape=jax.ShapeDtypeStruct((k,), jnp.float32),
    )(codebook)
    nb = n // _BN
    idx, mind2 = pl.pallas_call(
        _dist_body,
        grid=(nb,),
        in_specs=[
            pl.BlockSpec((_BN, d), lambda i: (i, 0)),
            pl.BlockSpec((k, d), lambda i: (0, 0)),
            pl.BlockSpec((k,), lambda i: (0,)),
        ],
        out_specs=[
            pl.BlockSpec((_BN,), lambda i: (i,)),
            pl.BlockSpec((_BN,), lambda i: (i,)),
        ],
        out_shape=[
            jax.ShapeDtypeStruct((n,), jnp.int32),
            jax.ShapeDtypeStruct((n,), jnp.float32),
        ],
        compiler_params=pltpu.CompilerParams(
            dimension_semantics=("parallel",)),
    )(z, codebook, csq)
    losssum = pl.pallas_call(
        functools.partial(_loss_body, denom=float(n * d)),
        out_shape=jax.ShapeDtypeStruct((1, 1), jnp.float32),
    )(mind2)
    return idx, losssum


def _sc_gather(codebook, indices):
    n = indices.shape[0]
    d = codebook.shape[1]
    idx2 = indices.reshape(1, n)
    mesh = plsc.VectorSubcoreMesh(
        core_axis_name="core", subcore_axis_name="subcore")

    @functools.partial(
        pl.kernel,
        out_type=jax.ShapeDtypeStruct((n, d), codebook.dtype),
        mesh=mesh)
    def gather_kernel(cb_hbm, i_hbm, o_hbm):
        def body(i_vmem, o_vmem):
            pltpu.sync_copy(cb_hbm.at[i_vmem.at[0]], o_vmem)

        pltpu.emit_pipeline(
            body,
            grid=(n // _GATHER_W,),
            in_specs=[pl.BlockSpec((1, _GATHER_W), index_map=lambda i: (0, i))],
            out_specs=[pl.BlockSpec((_GATHER_W, d), index_map=lambda i: (i, 0))],
            core_axis_name=("core", "subcore"),
            dimension_semantics=(pltpu.PARALLEL,),
        )(i_hbm, o_hbm)

    return gather_kernel(codebook, idx2)


def kernel(z, codebook):
    idx, losssum = _tc_dist_argmin(z, codebook)
    d = codebook.shape[1]
    if d < 128:
        cb_wide = jnp.pad(codebook, ((0, 0), (0, 128 - d)))
        quantized = _sc_gather(cb_wide, idx)[:, :d]
    else:
        quantized = _sc_gather(codebook, idx)
    return (quantized, idx, losssum[0, 0])
